# baseline (speedup 1.0000x reference)
import jax
import jax.numpy as jnp
from jax.experimental import pallas as pl
from jax.experimental.pallas import tpu as pltpu
from jax.experimental.pallas import tpu_sc as plsc

_G = 32
_S = 2

_SC_B = 2
_SC_CH = 256
_NC = 2
_NS = 16
_L = 16
_SC_W = 128
_SC_RB = 8
_SC_NG = _SC_W // _L


def _part_scan(ref, base_slab, ng, d):
    mv = jnp.full((_G, d), jnp.inf, jnp.float32)
    mi = jnp.zeros((_G, d), jnp.int32)
    for g in range(ng):
        v = ref[0, pl.ds(g * _G, _G), :]
        mask = v < mv
        mv = jnp.where(mask, v, mv)
        mi = jnp.where(mask, jnp.int32(base_slab + g), mi)
    return mv, mi


def _tc_body(*refs):
    x_refs, o_ref = refs[:-1], refs[-1]
    nh, d = x_refs[0].shape[1], x_refs[0].shape[2]
    ng = nh // _G
    mv, mi = _part_scan(x_refs[0], 0, ng, d)
    for s in range(1, _S):
        mvs, mis = _part_scan(x_refs[s], s * ng, ng, d)
        take = mvs < mv
        mv = jnp.where(take, mvs, mv)
        mi = jnp.where(take, mis, mi)
    m = jnp.min(mv, axis=0)
    track = jax.lax.broadcasted_iota(jnp.int32, (_G, d), 0)
    full = mi * _G + track
    big = jnp.int32(2**30)
    cand = jnp.where(mv == m[None], full, big)
    o_ref[0, 0, :] = jnp.min(cand, axis=0)


def _tc_argmin(x, nb):
    B, N, D = x.shape
    Nh = N // _S
    out = pl.pallas_call(
        _tc_body,
        grid=(nb,),
        in_specs=[
            pl.BlockSpec((1, Nh, D), lambda b, s=s: (b, s, 0))
            for s in range(_S)
        ],
        out_specs=pl.BlockSpec((1, 1, D), lambda b: (b, 0, 0)),
        out_shape=jax.ShapeDtypeStruct((nb, 1, D), jnp.int32),
        compiler_params=pltpu.CompilerParams(
            dimension_semantics=("arbitrary",),
        ),
    )(*([x] * _S))
    return out.reshape(nb, D)


def _sc_body(x_hbm, o_hbm, buf, loc_mv, loc_mi, sh_mv, sh_mi,
             mg_mv, mg_mi, sem0, sem1):
    nb = x_hbm.shape[0]
    n = x_hbm.shape[1]
    rows = n // _SC_RB
    nch = rows // _SC_CH
    cid = jax.lax.axis_index("c")
    sid = jax.lax.axis_index("s")
    b = nb - _SC_B + cid
    ch = sid // _SC_RB
    rb = sid % _SC_RB
    row0 = rb * rows
    col0 = ch * _SC_W
    sems = (sem0, sem1)

    def copy(k, do_start):
        cp = pltpu.make_async_copy(
            x_hbm.at[b, pl.ds(row0 + k * _SC_CH, _SC_CH),
                     pl.ds(col0, _SC_W)],
            buf.at[k % 2],
            sems[k % 2],
        )
        if do_start:
            cp.start()
        return cp

    copy(0, True)
    mv = [jnp.full((_L,), jnp.inf, jnp.float32) for _ in range(_SC_NG)]
    mi = [jnp.zeros((_L,), jnp.int32) for _ in range(_SC_NG)]
    for k in range(nch):
        if k + 1 < nch:
            copy(k + 1, True)
        copy(k, False).wait()
        bref = buf.at[k % 2]
        rv0 = jnp.zeros((_L,), jnp.int32) + (row0 + k * _SC_CH)

        def row_body(r, carry):
            mv, mi, rv = carry
            out_v, out_i = [], []
            for j in range(_SC_NG):
                v = bref[r, pl.ds(j * _L, _L)]
                mask = v < mv[j]
                out_v.append(jnp.where(mask, v, mv[j]))
                out_i.append(jnp.where(mask, rv, mi[j]))
            return out_v, out_i, rv + 1

        mv, mi, _ = jax.lax.fori_loop(0, _SC_CH, row_body, (mv, mi, rv0))
    for j in range(_SC_NG):
        loc_mv[pl.ds(j * _L, _L)] = mv[j]
        loc_mi[pl.ds(j * _L, _L)] = mi[j]
    pltpu.sync_copy(loc_mv, sh_mv.at[sid])
    pltpu.sync_copy(loc_mi, sh_mi.at[sid])
    plsc.subcore_barrier()

    @pl.when(sid < 2)
    def _merge():
        m = sid
        pltpu.sync_copy(sh_mv.at[pl.ds(m * _SC_RB, _SC_RB)], mg_mv)
        pltpu.sync_copy(sh_mi.at[pl.ds(m * _SC_RB, _SC_RB)], mg_mi)
        for j in range(_SC_NG):
            fv = mg_mv[0, pl.ds(j * _L, _L)]
            fi = mg_mi[0, pl.ds(j * _L, _L)]
            for k in range(1, _SC_RB):
                v = mg_mv[k, pl.ds(j * _L, _L)]
                i = mg_mi[k, pl.ds(j * _L, _L)]
                mask = v < fv
                fv = jnp.where(mask, v, fv)
                fi = jnp.where(mask, i, fi)
            loc_mi[pl.ds(j * _L, _L)] = fi
        d = _NC * _SC_W
        pltpu.sync_copy(loc_mi, o_hbm.at[pl.ds(cid * d + m * _SC_W, _SC_W)])


def _sc_argmin(x):
    _, N, D = x.shape
    Bsc = _SC_B
    mesh = plsc.VectorSubcoreMesh(
        core_axis_name="c", subcore_axis_name="s",
        num_cores=_NC, num_subcores=_NS,
    )
    k = pl.kernel(
        _sc_body,
        out_type=jax.ShapeDtypeStruct((Bsc * D,), jnp.int32),
        mesh=mesh,
        scratch_types=[
            pltpu.VMEM((2, _SC_CH, _SC_W), jnp.float32),
            pltpu.VMEM((_SC_W,), jnp.float32),
            pltpu.VMEM((_SC_W,), jnp.int32),
            pltpu.VMEM_SHARED((_NS, _SC_W), jnp.float32),
            pltpu.VMEM_SHARED((_NS, _SC_W), jnp.int32),
            pltpu.VMEM((_SC_RB, _SC_W), jnp.float32),
            pltpu.VMEM((_SC_RB, _SC_W), jnp.int32),
            pltpu.SemaphoreType.DMA,
            pltpu.SemaphoreType.DMA,
        ],
    )
    return k(x).reshape(Bsc, D)


def kernel(x):
    B, N, D = x.shape
    out_sc = _sc_argmin(x)
    out_tc = jnp.argmin(x[: B - _SC_B], axis=1).astype(jnp.int32)
    out = jnp.concatenate([out_tc, out_sc], axis=0)
    return out.astype(jnp.int64)


# speedup vs baseline: 1.3740x; 1.3740x over previous
import jax
import jax.numpy as jnp
from jax.experimental import pallas as pl
from jax.experimental.pallas import tpu as pltpu
from jax.experimental.pallas import tpu_sc as plsc

_G = 32
_S = 2

_SC_B = 2
_SC_CH = 256
_NC = 2
_NS = 16
_L = 16
_SC_W = 128
_SC_RB = 8
_SC_NG = _SC_W // _L


def _part_scan(ref, base_slab, ng, d):
    mv = jnp.full((_G, d), jnp.inf, jnp.float32)
    mi = jnp.zeros((_G, d), jnp.int32)
    for g in range(ng):
        v = ref[0, pl.ds(g * _G, _G), :]
        mask = v < mv
        mv = jnp.where(mask, v, mv)
        mi = jnp.where(mask, jnp.int32(base_slab + g), mi)
    return mv, mi


def _tc_body(*refs):
    x_refs, o_ref = refs[:-1], refs[-1]
    nh, d = x_refs[0].shape[1], x_refs[0].shape[2]
    ng = nh // _G
    mv, mi = _part_scan(x_refs[0], 0, ng, d)
    for s in range(1, _S):
        mvs, mis = _part_scan(x_refs[s], s * ng, ng, d)
        take = mvs < mv
        mv = jnp.where(take, mvs, mv)
        mi = jnp.where(take, mis, mi)
    m = jnp.min(mv, axis=0)
    track = jax.lax.broadcasted_iota(jnp.int32, (_G, d), 0)
    full = mi * _G + track
    big = jnp.int32(2**30)
    cand = jnp.where(mv == m[None], full, big)
    o_ref[0, 0, :] = jnp.min(cand, axis=0)


def _tc_argmin(x, nb):
    B, N, D = x.shape
    Nh = N // _S
    out = pl.pallas_call(
        _tc_body,
        grid=(nb,),
        in_specs=[
            pl.BlockSpec((1, Nh, D), lambda b, s=s: (b, s, 0))
            for s in range(_S)
        ],
        out_specs=pl.BlockSpec((1, 1, D), lambda b: (b, 0, 0)),
        out_shape=jax.ShapeDtypeStruct((nb, 1, D), jnp.int32),
        compiler_params=pltpu.CompilerParams(
            dimension_semantics=("arbitrary",),
            skip_device_barrier=True,
        ),
    )(*([x] * _S))
    return out.reshape(nb, D)


def _sc_body(x_hbm, o_hbm, buf, loc_mv, loc_mi, sh_mv, sh_mi,
             mg_mv, mg_mi, sem0, sem1):
    nb = x_hbm.shape[0]
    n = x_hbm.shape[1]
    rows = n // _SC_RB
    nch = rows // _SC_CH
    cid = jax.lax.axis_index("c")
    sid = jax.lax.axis_index("s")
    b = nb - _SC_B + cid
    ch = sid // _SC_RB
    rb = sid % _SC_RB
    row0 = rb * rows
    col0 = ch * _SC_W
    sems = (sem0, sem1)

    def copy(k, do_start):
        cp = pltpu.make_async_copy(
            x_hbm.at[b, pl.ds(row0 + k * _SC_CH, _SC_CH),
                     pl.ds(col0, _SC_W)],
            buf.at[k % 2],
            sems[k % 2],
        )
        if do_start:
            cp.start()
        return cp

    copy(0, True)
    mv = [jnp.full((_L,), jnp.inf, jnp.float32) for _ in range(_SC_NG)]
    mi = [jnp.zeros((_L,), jnp.int32) for _ in range(_SC_NG)]
    for k in range(nch):
        if k + 1 < nch:
            copy(k + 1, True)
        copy(k, False).wait()
        bref = buf.at[k % 2]
        rv0 = jnp.zeros((_L,), jnp.int32) + (row0 + k * _SC_CH)

        def row_body(r, carry):
            mv, mi, rv = carry
            out_v, out_i = [], []
            for j in range(_SC_NG):
                v = bref[r, pl.ds(j * _L, _L)]
                mask = v < mv[j]
                out_v.append(jnp.where(mask, v, mv[j]))
                out_i.append(jnp.where(mask, rv, mi[j]))
            return out_v, out_i, rv + 1

        mv, mi, _ = jax.lax.fori_loop(0, _SC_CH, row_body, (mv, mi, rv0))
    for j in range(_SC_NG):
        loc_mv[pl.ds(j * _L, _L)] = mv[j]
        loc_mi[pl.ds(j * _L, _L)] = mi[j]
    pltpu.sync_copy(loc_mv, sh_mv.at[sid])
    pltpu.sync_copy(loc_mi, sh_mi.at[sid])
    plsc.subcore_barrier()

    @pl.when(sid < 2)
    def _merge():
        m = sid
        pltpu.sync_copy(sh_mv.at[pl.ds(m * _SC_RB, _SC_RB)], mg_mv)
        pltpu.sync_copy(sh_mi.at[pl.ds(m * _SC_RB, _SC_RB)], mg_mi)
        for j in range(_SC_NG):
            fv = mg_mv[0, pl.ds(j * _L, _L)]
            fi = mg_mi[0, pl.ds(j * _L, _L)]
            for k in range(1, _SC_RB):
                v = mg_mv[k, pl.ds(j * _L, _L)]
                i = mg_mi[k, pl.ds(j * _L, _L)]
                mask = v < fv
                fv = jnp.where(mask, v, fv)
                fi = jnp.where(mask, i, fi)
            loc_mi[pl.ds(j * _L, _L)] = fi
        d = _NC * _SC_W
        pltpu.sync_copy(loc_mi, o_hbm.at[pl.ds(cid * d + m * _SC_W, _SC_W)])


def _sc_argmin(x):
    _, N, D = x.shape
    Bsc = _SC_B
    mesh = plsc.VectorSubcoreMesh(
        core_axis_name="c", subcore_axis_name="s",
        num_cores=_NC, num_subcores=_NS,
    )
    k = pl.kernel(
        _sc_body,
        out_type=jax.ShapeDtypeStruct((Bsc * D,), jnp.int32),
        mesh=mesh,
        scratch_types=[
            pltpu.VMEM((2, _SC_CH, _SC_W), jnp.float32),
            pltpu.VMEM((_SC_W,), jnp.float32),
            pltpu.VMEM((_SC_W,), jnp.int32),
            pltpu.VMEM_SHARED((_NS, _SC_W), jnp.float32),
            pltpu.VMEM_SHARED((_NS, _SC_W), jnp.int32),
            pltpu.VMEM((_SC_RB, _SC_W), jnp.float32),
            pltpu.VMEM((_SC_RB, _SC_W), jnp.int32),
            pltpu.SemaphoreType.DMA,
            pltpu.SemaphoreType.DMA,
        ],
        compiler_params=pltpu.CompilerParams(skip_device_barrier=True),
    )
    return k(x).reshape(Bsc, D)


def kernel(x):
    B, N, D = x.shape
    out_sc = _sc_argmin(x)
    out_tc = _tc_argmin(x, B - _SC_B)
    out = jnp.concatenate([out_tc, out_sc], axis=0)
    return out.astype(jnp.int64)


# speedup vs baseline: 1.9175x; 1.3955x over previous
import jax
import jax.numpy as jnp
from jax.experimental import pallas as pl
from jax.experimental.pallas import tpu as pltpu

_G = 32
_S = 2


def _part_scan(ref, base_slab, ng, d):
    mv = jnp.full((_G, d), jnp.inf, jnp.float32)
    mi = jnp.zeros((_G, d), jnp.int32)
    for g in range(ng):
        v = ref[0, pl.ds(g * _G, _G), :]
        mask = v < mv
        mv = jnp.where(mask, v, mv)
        mi = jnp.where(mask, jnp.int32(base_slab + g), mi)
    return mv, mi


def _argmin_body(*refs):
    x_refs, o_ref = refs[:-1], refs[-1]
    nh, d = x_refs[0].shape[1], x_refs[0].shape[2]
    ng = nh // _G
    mv, mi = _part_scan(x_refs[0], 0, ng, d)
    for s in range(1, _S):
        mvs, mis = _part_scan(x_refs[s], s * ng, ng, d)
        take = mvs < mv
        mv = jnp.where(take, mvs, mv)
        mi = jnp.where(take, mis, mi)
    m = jnp.min(mv, axis=0)
    track = jax.lax.broadcasted_iota(jnp.int32, (_G, d), 0)
    full = mi * _G + track
    big = jnp.int32(2**30)
    cand = jnp.where(mv == m[None], full, big)
    o_ref[0, 0, :] = jnp.min(cand, axis=0)


def kernel(x):
    B, N, D = x.shape
    Nh = N // _S
    out = pl.pallas_call(
        _argmin_body,
        grid=(B,),
        in_specs=[
            pl.BlockSpec((1, Nh, D), lambda b, s=s: (b, s, 0))
            for s in range(_S)
        ],
        out_specs=pl.BlockSpec((1, 1, D), lambda b: (b, 0, 0)),
        out_shape=jax.ShapeDtypeStruct((B, 1, D), jnp.int32),
        compiler_params=pltpu.CompilerParams(
            dimension_semantics=("arbitrary",),
        ),
    )(*([x] * _S))
    return out.reshape(B, D).astype(jnp.int64)
